# baseline (speedup 1.0000x reference)
import functools

import jax
import jax.numpy as jnp
from jax import lax
from jax.experimental import pallas as pl
from jax.experimental.pallas import tpu as pltpu
from jax.experimental.pallas import tpu_sc as plsc

N_DENSE = 13
N_SPARSE = 26
VOCAB = 1_000_000
BATCH = 16384

NC, NS, L = 2, 16, 16
NW = NC * NS
BPW = BATCH // NW
NCHUNK = BPW // L
GCH = 128
KG = BPW // GCH

_mesh = plsc.VectorSubcoreMesh(core_axis_name="c", subcore_axis_name="s")


@functools.partial(
    pl.kernel,
    out_type=jax.ShapeDtypeStruct((BATCH,), jnp.float32),
    mesh=_mesh,
    scratch_types=[
        pltpu.VMEM((N_SPARSE, BPW), jnp.float32),
        pltpu.VMEM((N_DENSE, BPW), jnp.float32),
        pltpu.VMEM((N_DENSE, L), jnp.float32),
        pltpu.VMEM((N_SPARSE, BPW), jnp.int32),
        pltpu.VMEM((N_SPARSE, BPW), jnp.float32),
        pltpu.VMEM((BPW,), jnp.float32),
        pltpu.SemaphoreType.DMA,
    ],
)
def _linear_sc(xs_hbm, xd_hbm, w_hbm, table_hbm, out_hbm,
               xs_v, xd_v, w_v, idx_v, rows_v, out_v, sem):
    wid = lax.axis_index("s") * NC + lax.axis_index("c")
    base = wid * BPW

    pltpu.sync_copy(xs_hbm.at[:, pl.ds(base, BPW)], xs_v)
    pltpu.sync_copy(xd_hbm.at[:, pl.ds(base, BPW)], xd_v)
    pltpu.sync_copy(w_hbm, w_v)

    def build_body(f, carry):
        off = f * VOCAB
        for c in range(NCHUNK):
            v = xs_v[f, pl.ds(c * L, L)]
            idx_v[f, pl.ds(c * L, L)] = v.astype(jnp.int32) + off
        return carry

    lax.fori_loop(0, N_SPARSE, build_body, 0)

    for f in range(N_SPARSE):
        for k in range(KG):
            pltpu.async_copy(
                table_hbm.at[idx_v.at[f, pl.ds(k * GCH, GCH)]],
                rows_v.at[f, pl.ds(k * GCH, GCH)],
                sem,
            ).wait()

    def sum_body(c, carry):
        o = c * L
        acc = rows_v[0, pl.ds(o, L)]
        for f in range(1, N_SPARSE):
            acc = acc + rows_v[f, pl.ds(o, L)]
        for k in range(N_DENSE):
            acc = acc + xd_v[k, pl.ds(o, L)] * w_v[k, :]
        out_v[pl.ds(o, L)] = acc
        return carry

    lax.fori_loop(0, NCHUNK, sum_body, 0)
    pltpu.sync_copy(out_v, out_hbm.at[pl.ds(base, BPW)])


def kernel(X, emb_tables, dense_weight):
    xs = X[:, N_DENSE:].T
    xd = X[:, :N_DENSE].T
    w = jnp.broadcast_to(dense_weight.reshape(N_DENSE, 1), (N_DENSE, L))
    table = emb_tables.reshape(N_SPARSE * VOCAB)
    out = _linear_sc(xs, xd, w, table)
    return out.reshape(BATCH, 1)


# speedup vs baseline: 1.0261x; 1.0261x over previous
import functools

import jax
import jax.numpy as jnp
from jax import lax
from jax.experimental import pallas as pl
from jax.experimental.pallas import tpu as pltpu
from jax.experimental.pallas import tpu_sc as plsc

N_DENSE = 13
N_SPARSE = 26
VOCAB = 1_000_000
BATCH = 16384

NC, NS, L = 2, 16, 16
NW = NC * NS
BPW = BATCH // NW
NCHUNK = BPW // L
GCH = 128
KG = BPW // GCH

_mesh = plsc.VectorSubcoreMesh(core_axis_name="c", subcore_axis_name="s")


@functools.partial(
    pl.kernel,
    out_type=jax.ShapeDtypeStruct((BATCH,), jnp.float32),
    mesh=_mesh,
    scratch_types=[
        pltpu.VMEM((N_SPARSE, BPW), jnp.float32),
        pltpu.VMEM((N_DENSE, BPW), jnp.float32),
        pltpu.VMEM((N_DENSE, L), jnp.float32),
        pltpu.VMEM((N_SPARSE, BPW), jnp.int32),
        pltpu.VMEM((N_SPARSE, BPW), jnp.float32),
        pltpu.VMEM((BPW,), jnp.float32),
        pltpu.SemaphoreType.DMA,
    ],
)
def _linear_sc(xs_hbm, xd_hbm, w_hbm, table_hbm, out_hbm,
               xs_v, xd_v, w_v, idx_v, rows_v, out_v, sem):
    wid = lax.axis_index("s") * NC + lax.axis_index("c")
    base = wid * BPW

    pltpu.sync_copy(xs_hbm.at[:, pl.ds(base, BPW)], xs_v)
    pltpu.sync_copy(xd_hbm.at[:, pl.ds(base, BPW)], xd_v)
    pltpu.sync_copy(w_hbm, w_v)

    def build_body(f, carry):
        off = f * VOCAB
        for c in range(NCHUNK):
            v = xs_v[f, pl.ds(c * L, L)]
            idx_v[f, pl.ds(c * L, L)] = v.astype(jnp.int32) + off
        return carry

    lax.fori_loop(0, N_SPARSE, build_body, 0)

    handles = []
    for f in range(N_SPARSE):
        for k in range(KG):
            handles.append(pltpu.async_copy(
                table_hbm.at[idx_v.at[f, pl.ds(k * GCH, GCH)]],
                rows_v.at[f, pl.ds(k * GCH, GCH)],
                sem,
            ))
    for h in handles:
        h.wait()

    def sum_body(c, carry):
        o = c * L
        acc = rows_v[0, pl.ds(o, L)]
        for f in range(1, N_SPARSE):
            acc = acc + rows_v[f, pl.ds(o, L)]
        for k in range(N_DENSE):
            acc = acc + xd_v[k, pl.ds(o, L)] * w_v[k, :]
        out_v[pl.ds(o, L)] = acc
        return carry

    lax.fori_loop(0, NCHUNK, sum_body, 0)
    pltpu.sync_copy(out_v, out_hbm.at[pl.ds(base, BPW)])


def kernel(X, emb_tables, dense_weight):
    xs = X[:, N_DENSE:].T
    xd = X[:, :N_DENSE].T
    w = jnp.broadcast_to(dense_weight.reshape(N_DENSE, 1), (N_DENSE, L))
    table = emb_tables.reshape(N_SPARSE * VOCAB)
    out = _linear_sc(xs, xd, w, table)
    return out.reshape(BATCH, 1)


# speedup vs baseline: 1.0271x; 1.0010x over previous
import functools

import jax
import jax.numpy as jnp
from jax import lax
from jax.experimental import pallas as pl
from jax.experimental.pallas import tpu as pltpu
from jax.experimental.pallas import tpu_sc as plsc

N_DENSE = 13
N_SPARSE = 26
VOCAB = 1_000_000
BATCH = 16384

NC, NS, L = 2, 16, 16
NW = NC * NS
BPW = BATCH // NW
NCHUNK = BPW // L
GCH = 128
KG = BPW // GCH

_mesh = plsc.VectorSubcoreMesh(core_axis_name="c", subcore_axis_name="s")


@functools.partial(
    pl.kernel,
    out_type=jax.ShapeDtypeStruct((BATCH,), jnp.float32),
    mesh=_mesh,
    scratch_types=[
        pltpu.VMEM((N_SPARSE, BPW), jnp.float32),
        pltpu.VMEM((N_DENSE, BPW), jnp.float32),
        pltpu.VMEM((N_DENSE, L), jnp.float32),
        pltpu.VMEM((N_SPARSE, BPW), jnp.int32),
        pltpu.VMEM((N_SPARSE, BPW), jnp.float32),
        pltpu.VMEM((BPW,), jnp.float32),
        pltpu.SemaphoreType.DMA,
    ],
    compiler_params=pltpu.CompilerParams(use_tc_tiling_on_sc=False),
)
def _linear_sc(xs_hbm, xd_hbm, w_hbm, table_hbm, out_hbm,
               xs_v, xd_v, w_v, idx_v, rows_v, out_v, sem):
    wid = lax.axis_index("s") * NC + lax.axis_index("c")
    base = wid * BPW

    pltpu.sync_copy(xs_hbm.at[:, pl.ds(base, BPW)], xs_v)
    pltpu.sync_copy(xd_hbm.at[:, pl.ds(base, BPW)], xd_v)
    pltpu.sync_copy(w_hbm, w_v)

    def build_body(f, carry):
        for c in range(NCHUNK):
            v = xs_v[f, pl.ds(c * L, L)]
            idx_v[f, pl.ds(c * L, L)] = v.astype(jnp.int32)
        return carry

    lax.fori_loop(0, N_SPARSE, build_body, 0)

    handles = []
    for f in range(N_SPARSE):
        for k in range(KG):
            handles.append(pltpu.async_copy(
                table_hbm.at[f].at[idx_v.at[f, pl.ds(k * GCH, GCH)]],
                rows_v.at[f, pl.ds(k * GCH, GCH)],
                sem,
            ))
    for h in handles:
        h.wait()

    def sum_body(c, carry):
        o = c * L
        acc = rows_v[0, pl.ds(o, L)]
        for f in range(1, N_SPARSE):
            acc = acc + rows_v[f, pl.ds(o, L)]
        for k in range(N_DENSE):
            acc = acc + xd_v[k, pl.ds(o, L)] * w_v[k, :]
        out_v[pl.ds(o, L)] = acc
        return carry

    lax.fori_loop(0, NCHUNK, sum_body, 0)
    pltpu.sync_copy(out_v, out_hbm.at[pl.ds(base, BPW)])


def kernel(X, emb_tables, dense_weight):
    xs = X[:, N_DENSE:].T
    xd = X[:, :N_DENSE].T
    w = jnp.broadcast_to(dense_weight.reshape(N_DENSE, 1), (N_DENSE, L))
    out = _linear_sc(xs, xd, w, emb_tables[:, :, 0])
    return out.reshape(BATCH, 1)


# speedup vs baseline: 1.8427x; 1.7940x over previous
import functools

import jax
import jax.numpy as jnp
from jax import lax
from jax.experimental import pallas as pl
from jax.experimental.pallas import tpu as pltpu
from jax.experimental.pallas import tpu_sc as plsc

N_DENSE = 13
N_SPARSE = 26
VOCAB = 1_000_000
BATCH = 16384

NC, NS, L = 2, 16, 16
NW = NC * NS
BPW = BATCH // NW
NCHUNK = BPW // L
GCH = 128
KG = BPW // GCH

_mesh = plsc.VectorSubcoreMesh(core_axis_name="c", subcore_axis_name="s")


@functools.partial(
    pl.kernel,
    out_type=jax.ShapeDtypeStruct((BATCH,), jnp.float32),
    mesh=_mesh,
    scratch_types=[
        pltpu.VMEM((N_SPARSE, BPW), jnp.float32),
        pltpu.VMEM((N_DENSE, BPW), jnp.float32),
        pltpu.VMEM((N_DENSE, L), jnp.float32),
        pltpu.VMEM((N_SPARSE, BPW), jnp.int32),
        pltpu.VMEM((N_SPARSE, BPW), jnp.float32),
        pltpu.VMEM((BPW,), jnp.float32),
        pltpu.SemaphoreType.DMA,
    ],
)
def _linear_sc(xs_hbm, xd_hbm, w_hbm, table_hbm, out_hbm,
               xs_v, xd_v, w_v, idx_v, rows_v, out_v, sem):
    wid = lax.axis_index("s") * NC + lax.axis_index("c")
    base = wid * BPW

    pltpu.sync_copy(xs_hbm.at[:, pl.ds(base, BPW)], xs_v)
    pltpu.sync_copy(xd_hbm.at[:, pl.ds(base, BPW)], xd_v)
    pltpu.sync_copy(w_hbm, w_v)

    def build_body(f, carry):
        off = f * VOCAB
        for c in range(NCHUNK):
            v = xs_v[f, pl.ds(c * L, L)]
            idx_v[f, pl.ds(c * L, L)] = v.astype(jnp.int32) + off
        return carry

    lax.fori_loop(0, N_SPARSE, build_body, 0)

    handles = []
    for f in range(N_SPARSE):
        for k in range(KG):
            handles.append(pltpu.async_copy(
                table_hbm.at[idx_v.at[f, pl.ds(k * GCH, GCH)]],
                rows_v.at[f, pl.ds(k * GCH, GCH)],
                sem,
            ))
    for h in handles:
        h.wait()

    def sum_body(c, carry):
        o = c * L
        acc = rows_v[0, pl.ds(o, L)]
        for f in range(1, N_SPARSE):
            acc = acc + rows_v[f, pl.ds(o, L)]
        for k in range(N_DENSE):
            acc = acc + xd_v[k, pl.ds(o, L)] * w_v[k, :]
        out_v[pl.ds(o, L)] = acc
        return carry

    lax.fori_loop(0, NCHUNK, sum_body, 0)
    pltpu.sync_copy(out_v, out_hbm.at[pl.ds(base, BPW)])


def kernel(X, emb_tables, dense_weight):
    xs = X[:, N_DENSE:].T
    xd = X[:, :N_DENSE].T
    w = jnp.broadcast_to(dense_weight.reshape(N_DENSE, 1), (N_DENSE, L))
    table = jnp.concatenate([emb_tables[f, :, 0] for f in range(N_SPARSE)])
    out = _linear_sc(xs, xd, w, table)
    return out.reshape(BATCH, 1)


# speedup vs baseline: 5.1595x; 2.8000x over previous
import functools

import jax
import jax.numpy as jnp
from jax import lax
from jax.experimental import pallas as pl
from jax.experimental.pallas import tpu as pltpu
from jax.experimental.pallas import tpu_sc as plsc

N_DENSE = 13
N_SPARSE = 26
VOCAB = 1_000_000
BATCH = 16384

NC, NS, L = 2, 16, 16
NW = NC * NS
BPW = BATCH // NW
NCHUNK = BPW // L
GCH = 128
KG = BPW // GCH

_mesh = plsc.VectorSubcoreMesh(core_axis_name="c", subcore_axis_name="s")


@functools.partial(
    pl.kernel,
    out_type=jax.ShapeDtypeStruct((BATCH,), jnp.float32),
    mesh=_mesh,
    scratch_types=[
        pltpu.VMEM((N_SPARSE, BPW), jnp.float32),
        pltpu.VMEM((N_DENSE, BPW), jnp.float32),
        pltpu.VMEM((N_DENSE, L), jnp.float32),
        pltpu.VMEM((N_SPARSE, BPW), jnp.int32),
        pltpu.VMEM((N_SPARSE, BPW), jnp.float32),
        pltpu.VMEM((BPW,), jnp.float32),
        pltpu.SemaphoreType.DMA,
    ],
)
def _linear_sc(xs_hbm, xd_hbm, w_hbm, t0, t1, t2, t3, t4, t5, t6, t7, t8, t9, t10, t11, t12, t13, t14, t15, t16, t17, t18, t19, t20, t21, t22, t23, t24, t25, out_hbm,
               xs_v, xd_v, w_v, idx_v, rows_v, out_v, sem):
    tables = [t0, t1, t2, t3, t4, t5, t6, t7, t8, t9, t10, t11, t12, t13, t14, t15, t16, t17, t18, t19, t20, t21, t22, t23, t24, t25]
    wid = lax.axis_index("s") * NC + lax.axis_index("c")
    base = wid * BPW

    pltpu.sync_copy(xs_hbm.at[:, pl.ds(base, BPW)], xs_v)
    pltpu.sync_copy(xd_hbm.at[:, pl.ds(base, BPW)], xd_v)
    pltpu.sync_copy(w_hbm, w_v)

    def build_body(f, carry):
        for c in range(NCHUNK):
            v = xs_v[f, pl.ds(c * L, L)]
            idx_v[f, pl.ds(c * L, L)] = v.astype(jnp.int32)
        return carry

    lax.fori_loop(0, N_SPARSE, build_body, 0)

    handles = []
    for f in range(N_SPARSE):
        for k in range(KG):
            handles.append(pltpu.async_copy(
                tables[f].at[idx_v.at[f, pl.ds(k * GCH, GCH)]],
                rows_v.at[f, pl.ds(k * GCH, GCH)],
                sem,
            ))
    for h in handles:
        h.wait()

    def sum_body(c, carry):
        o = c * L
        acc = rows_v[0, pl.ds(o, L)]
        for f in range(1, N_SPARSE):
            acc = acc + rows_v[f, pl.ds(o, L)]
        for k in range(N_DENSE):
            acc = acc + xd_v[k, pl.ds(o, L)] * w_v[k, :]
        out_v[pl.ds(o, L)] = acc
        return carry

    lax.fori_loop(0, NCHUNK, sum_body, 0)
    pltpu.sync_copy(out_v, out_hbm.at[pl.ds(base, BPW)])


def kernel(X, emb_tables, dense_weight):
    xs = X[:, N_DENSE:].T
    xd = X[:, :N_DENSE].T
    w = jnp.broadcast_to(dense_weight.reshape(N_DENSE, 1), (N_DENSE, L))
    parts = [emb_tables[f, :, 0] for f in range(N_SPARSE)]
    out = _linear_sc(xs, xd, w, *parts)
    return out.reshape(BATCH, 1)
